# baseline (speedup 1.0000x reference)
import dataclasses
import functools

import jax
import jax.numpy as jnp
from jax import lax
from jax.experimental import pallas as pl
from jax.experimental.pallas import tpu as pltpu
from jax.experimental.pallas import tpu_sc as plsc

_B, _C = 16, 4096
_L = 16
_HALF = _C // 2


def _frames_sc_kernel(x_hbm, g_hbm, lens_hbm, out_hbm, ybuf, lens_v, obuf, sem):
    wid = lax.axis_index("s") * 2 + lax.axis_index("c")
    row = wid // 2
    h = wid % 2

    cp_x = pltpu.async_copy(x_hbm.at[row], ybuf.at[pl.ds(0, _C)], sem)
    cp_g = pltpu.async_copy(g_hbm.at[row], ybuf.at[pl.ds(_C, _C)], sem)
    cp_l = pltpu.async_copy(lens_hbm, lens_v, sem)
    cp_x.wait()
    cp_g.wait()
    cp_l.wait()

    iota = lax.iota(jnp.int32, _L)
    s_vec = plsc.load_gather(lens_v, [jnp.full((_L,), row, jnp.int32)])
    base = s_vec + h * _HALF + iota
    for j in range(_HALF // _L):
        out16 = plsc.load_gather(ybuf, [base + (j * _L)])
        obuf[pl.ds(j * _L, _L)] = out16

    pltpu.async_copy(obuf, out_hbm.at[row, pl.ds(h * _HALF, _HALF)], sem).wait()


def kernel(x, ragged_dense, row_lengths):
    lens = row_lengths.astype(jnp.int32)
    mesh = plsc.VectorSubcoreMesh(core_axis_name="c", subcore_axis_name="s")
    cp = pltpu.CompilerParams()
    if "needs_layout_passes" in pltpu.CompilerParams.__dataclass_fields__:
        cp = dataclasses.replace(cp, needs_layout_passes=False)
    sc_call = pl.kernel(
        _frames_sc_kernel,
        out_type=jax.ShapeDtypeStruct((_B, _C), jnp.float32),
        mesh=mesh,
        scratch_types=[
            pltpu.VMEM((2 * _C,), jnp.float32),
            pltpu.VMEM((_B,), jnp.int32),
            pltpu.VMEM((_HALF,), jnp.float32),
            pltpu.SemaphoreType.DMA,
        ],
        compiler_params=cp,
    )
    out = sc_call(x, ragged_dense, lens)
    return out, lens[:, None]


# speedup vs baseline: 1.0233x; 1.0233x over previous
import dataclasses
import functools

import jax
import jax.numpy as jnp
from jax import lax
from jax.experimental import pallas as pl
from jax.experimental.pallas import tpu as pltpu
from jax.experimental.pallas import tpu_sc as plsc

_B, _C = 16, 4096
_L = 16
_HALF = _C // 2


def _frames_sc_kernel(x_hbm, g_hbm, lens_hbm, out_hbm, ybuf, lens_v, obuf, sem):
    row = lax.axis_index("s")

    cp_x = pltpu.async_copy(x_hbm.at[row], ybuf.at[pl.ds(0, _C)], sem)
    cp_g = pltpu.async_copy(g_hbm.at[row], ybuf.at[pl.ds(_C, _C)], sem)
    cp_l = pltpu.async_copy(lens_hbm, lens_v, sem)
    cp_l.wait()
    cp_x.wait()
    cp_g.wait()

    iota = lax.iota(jnp.int32, _L)
    s_vec = plsc.load_gather(lens_v, [jnp.full((_L,), row, jnp.int32)])
    base = s_vec + iota
    for j in range(_C // _L):
        out16 = plsc.load_gather(ybuf, [base + (j * _L)])
        obuf[pl.ds(j * _L, _L)] = out16

    pltpu.async_copy(obuf, out_hbm.at[row], sem).wait()


def kernel(x, ragged_dense, row_lengths):
    lens = row_lengths.astype(jnp.int32)
    mesh = plsc.VectorSubcoreMesh(core_axis_name="c", subcore_axis_name="s", num_cores=1)
    cp = pltpu.CompilerParams()
    if "needs_layout_passes" in pltpu.CompilerParams.__dataclass_fields__:
        cp = dataclasses.replace(cp, needs_layout_passes=False)
    sc_call = pl.kernel(
        _frames_sc_kernel,
        out_type=jax.ShapeDtypeStruct((_B, _C), jnp.float32),
        mesh=mesh,
        scratch_types=[
            pltpu.VMEM((2 * _C,), jnp.float32),
            pltpu.VMEM((_B,), jnp.int32),
            pltpu.VMEM((_C,), jnp.float32),
            pltpu.SemaphoreType.DMA,
        ],
        compiler_params=cp,
    )
    out = sc_call(x, ragged_dense, lens)
    return out, lens[:, None]


# speedup vs baseline: 1.0309x; 1.0075x over previous
import dataclasses
import functools

import jax
import jax.numpy as jnp
from jax import lax
from jax.experimental import pallas as pl
from jax.experimental.pallas import tpu as pltpu
from jax.experimental.pallas import tpu_sc as plsc

_B, _C = 16, 4096
_L = 16
_HALF = _C // 2


def _frames_sc_kernel(x_hbm, g_hbm, lens_hbm, out_hbm, ybuf, lens_v, obuf,
                      sem, sem_l, sem_out):
    row = lax.axis_index("s")

    cp_x = pltpu.async_copy(x_hbm.at[row], ybuf.at[pl.ds(0, _C)], sem)
    cp_g = pltpu.async_copy(g_hbm.at[row], ybuf.at[pl.ds(_C, _C)], sem)
    cp_l = pltpu.async_copy(lens_hbm, lens_v, sem_l)
    cp_l.wait()

    iota = lax.iota(jnp.int32, _L)
    s_vec = plsc.load_gather(lens_v, [jnp.full((_L,), row, jnp.int32)])
    base = s_vec + iota
    cp_x.wait()
    cp_g.wait()

    n_chunks = 4
    chunk = _C // n_chunks
    cp_out = []
    for c in range(n_chunks):
        for j in range(c * (chunk // _L), (c + 1) * (chunk // _L)):
            out16 = plsc.load_gather(ybuf, [base + (j * _L)])
            obuf[pl.ds(j * _L, _L)] = out16
        cp_out.append(
            pltpu.async_copy(
                obuf.at[pl.ds(c * chunk, chunk)],
                out_hbm.at[row, pl.ds(c * chunk, chunk)],
                sem_out,
            )
        )
    for cp in cp_out:
        cp.wait()


def kernel(x, ragged_dense, row_lengths):
    lens = row_lengths.astype(jnp.int32)
    mesh = plsc.VectorSubcoreMesh(core_axis_name="c", subcore_axis_name="s", num_cores=1)
    cp = pltpu.CompilerParams()
    if "needs_layout_passes" in pltpu.CompilerParams.__dataclass_fields__:
        cp = dataclasses.replace(cp, needs_layout_passes=False)
    sc_call = pl.kernel(
        _frames_sc_kernel,
        out_type=jax.ShapeDtypeStruct((_B, _C), jnp.float32),
        mesh=mesh,
        scratch_types=[
            pltpu.VMEM((2 * _C,), jnp.float32),
            pltpu.VMEM((_B,), jnp.int32),
            pltpu.VMEM((_C,), jnp.float32),
            pltpu.SemaphoreType.DMA,
            pltpu.SemaphoreType.DMA,
            pltpu.SemaphoreType.DMA,
        ],
        compiler_params=cp,
    )
    out = sc_call(x, ragged_dense, lens)
    return out, lens[:, None]


# speedup vs baseline: 1.0792x; 1.0469x over previous
import dataclasses
import functools

import jax
import jax.numpy as jnp
from jax import lax
from jax.experimental import pallas as pl
from jax.experimental.pallas import tpu as pltpu
from jax.experimental.pallas import tpu_sc as plsc

_B, _C = 16, 4096
_L = 16
_HALF = _C // 2


def _frames_sc_kernel(x_hbm, g_hbm, lens_hbm, out_hbm, ybuf, lens_v, obuf,
                      sem, sem_l, sem_out):
    row = lax.axis_index("s")

    cp_x = pltpu.async_copy(x_hbm.at[row], ybuf.at[pl.ds(0, _C)], sem)
    cp_g = pltpu.async_copy(g_hbm.at[row], ybuf.at[pl.ds(_C, _C)], sem)
    cp_l = pltpu.async_copy(lens_hbm, lens_v, sem_l)
    cp_l.wait()

    iota = lax.iota(jnp.int32, _L)
    s_vec = plsc.load_gather(lens_v, [jnp.full((_L,), row, jnp.int32)])
    base = s_vec + iota
    cp_x.wait()
    cp_g.wait()

    n_chunks = 4
    chunk = _C // n_chunks
    cp_out = []
    for c in range(n_chunks):
        for g in range(c * (chunk // _L) // 4, (c + 1) * (chunk // _L) // 4):
            js = [4 * g + k for k in range(4)]
            vals = [plsc.load_gather(ybuf, [base + (j * _L)]) for j in js]
            for j, v in zip(js, vals):
                obuf[pl.ds(j * _L, _L)] = v
        cp_out.append(
            pltpu.async_copy(
                obuf.at[pl.ds(c * chunk, chunk)],
                out_hbm.at[row, pl.ds(c * chunk, chunk)],
                sem_out,
            )
        )
    for cp in cp_out:
        cp.wait()


def kernel(x, ragged_dense, row_lengths):
    lens = row_lengths.astype(jnp.int32)
    mesh = plsc.VectorSubcoreMesh(core_axis_name="c", subcore_axis_name="s", num_cores=1)
    cp = pltpu.CompilerParams()
    if "needs_layout_passes" in pltpu.CompilerParams.__dataclass_fields__:
        cp = dataclasses.replace(cp, needs_layout_passes=False)
    sc_call = pl.kernel(
        _frames_sc_kernel,
        out_type=jax.ShapeDtypeStruct((_B, _C), jnp.float32),
        mesh=mesh,
        scratch_types=[
            pltpu.VMEM((2 * _C,), jnp.float32),
            pltpu.VMEM((_B,), jnp.int32),
            pltpu.VMEM((_C,), jnp.float32),
            pltpu.SemaphoreType.DMA,
            pltpu.SemaphoreType.DMA,
            pltpu.SemaphoreType.DMA,
        ],
        compiler_params=cp,
    )
    out = sc_call(x, ragged_dense, lens)
    return out, lens[:, None]
